# baseline (speedup 1.0000x reference)
import functools

import jax
import jax.numpy as jnp
from jax import lax
from jax.experimental import pallas as pl
from jax.experimental.pallas import tpu as pltpu
from jax.experimental.pallas import tpu_sc as plsc

_NM, _NT, _NF = 400000, 10000, 100000
_N = _NM + _NT + _NF
_NBY = 512
_NBINS = 512 * 512
_TD = 0.9
_BLK = 128
_NBC = 8
_CHK = _NBC * _BLK
_NW = 32
_LPT = _NBINS // 16


def _sc_body(pos_hbm, nsx_hbm, nsy_hbm, out_hbm,
             xb0, yb0, sxb0, syb0, xb1, yb1, sxb1, syb1,
             idx0, val0, idx1, val1, idx2, val2, idx3, val3, zb, map_sh,
             insem0, insem1, scsem0, scsem1, scsem2, scsem3):
    cid = lax.axis_index("c")
    sid = lax.axis_index("s")
    wid = sid * 2 + cid
    inbufs = ((xb0, yb0, sxb0, syb0), (xb1, yb1, sxb1, syb1))
    stag = ((idx0, val0), (idx1, val1), (idx2, val2), (idx3, val3))
    insems = (insem0, insem1)
    scsems = (scsem0, scsem1, scsem2, scsem3)

    def zero_body(i, carry):
        zb[pl.ds(i * 16, 16)] = jnp.zeros((16,), jnp.float32)
        return carry

    lax.fori_loop(0, _LPT // 16, zero_body, 0)
    pltpu.sync_copy(zb, map_sh.at[pl.ds(sid * _LPT, _LPT)])
    plsc.subcore_barrier()

    def input_start(par, off, n):
        xb, yb, sxb, syb = inbufs[par]
        sem = insems[par]
        pltpu.async_copy(pos_hbm.at[pl.ds(off, n)], xb.at[pl.ds(0, n)], sem)
        pltpu.async_copy(pos_hbm.at[pl.ds(_N + off, n)], yb.at[pl.ds(0, n)], sem)
        pltpu.async_copy(nsx_hbm.at[pl.ds(off, n)], sxb.at[pl.ds(0, n)], sem)
        pltpu.async_copy(nsy_hbm.at[pl.ds(off, n)], syb.at[pl.ds(0, n)], sem)

    def input_wait(par, off, n):
        xb, yb, sxb, syb = inbufs[par]
        sem = insems[par]
        pltpu.make_async_copy(
            pos_hbm.at[pl.ds(off, n)], xb.at[pl.ds(0, n)], sem).wait()
        pltpu.make_async_copy(
            pos_hbm.at[pl.ds(_N + off, n)], yb.at[pl.ds(0, n)], sem).wait()
        pltpu.make_async_copy(
            nsx_hbm.at[pl.ds(off, n)], sxb.at[pl.ds(0, n)], sem).wait()
        pltpu.make_async_copy(
            nsy_hbm.at[pl.ds(off, n)], syb.at[pl.ds(0, n)], sem).wait()

    def drain(sp, nrows):
        idxb, valb = stag[sp]
        sem = scsems[sp]
        for t in range(nrows):
            pltpu.make_async_copy(
                valb.at[t], map_sh.at[idxb.at[t]], sem).wait()

    def fire(sp, nrows):
        idxb, valb = stag[sp]
        sem = scsems[sp]
        for t in range(nrows):
            pltpu.async_copy(
                valb.at[t], map_sh.at[idxb.at[t]], sem, add=True)

    def overlaps(lo0, ext0, ext1, K, scale, mask):
        out = []
        for i in range(K):
            lo = lo0 + (2.0 * i)
            o = jnp.maximum(jnp.minimum(ext1, lo + 2.0) - jnp.maximum(ext0, lo),
                            0.0)
            if scale != 1.0:
                o = o * scale
            if mask is not None:
                o = jnp.where(mask, o, 0.0)
            out.append(o)
        return out

    def do_range(start, count, K, scale):
        nblocks = (count + _BLK - 1) // _BLK
        nchunks = (nblocks + _NBC - 1) // _NBC
        range_end = start + count
        masked = count % _CHK != 0
        max_k = (nchunks + _NW - 1) // _NW
        K2 = K * K

        def chunk_off(u):
            off = start + u * _CHK
            if masked:
                off = jnp.minimum(off, range_end - _CHK)
            return off

        def stage_vreg(par, sp, boff, abs_off, off_orig, v):
            xb, yb, sxb, syb = inbufs[par]
            idxb, valb = stag[sp]
            sl = pl.ds(boff + v * 16, 16)
            x = xb[sl]
            y = yb[sl]
            xe = x + sxb[sl]
            ye = y + syb[sl]
            bx0 = (x * 0.5).astype(jnp.int32)
            by0 = (y * 0.5).astype(jnp.int32)
            blx = bx0.astype(jnp.float32) * 2.0
            bly = by0.astype(jnp.float32) * 2.0
            ibase = bx0 * _NBY + by0
            if masked:
                lane = lax.iota(jnp.int32, 16)
                gid = abs_off + v * 16 + lane
                mask = jnp.logical_and(gid >= off_orig, gid < range_end)
                ibase = jnp.where(mask, ibase, v * 16 + lane)
            else:
                mask = None
            ox = overlaps(blx, x, xe, K, scale, mask)
            oy = overlaps(bly, y, ye, K, 1.0, None)
            for i in range(K):
                for j in range(K):
                    t = i * K + j
                    idxb[t, pl.ds(v * 16, 16)] = ibase + (i * _NBY + j)
                    valb[t, pl.ds(v * 16, 16)] = ox[i] * oy[j]

        def process_chunk(par, k, u):
            coff = chunk_off(u)
            input_wait(par, coff, _CHK)
            for c in range(_NBC):
                sp = c & 3
                off_orig = start + (u * _NBC + c) * _BLK
                if masked:
                    boff = jnp.minimum(off_orig, range_end - _BLK) - coff
                else:
                    boff = c * _BLK
                if c < 4:
                    @pl.when(k > 0)
                    def _():
                        drain(sp, K2)
                else:
                    drain(sp, K2)

                def vbody(v, carry):
                    stage_vreg(par, sp, boff, coff + boff, off_orig, v)
                    return carry

                lax.fori_loop(0, _BLK // 16, vbody, 0)
                fire(sp, K2)

            @pl.when(u + 2 * _NW < nchunks)
            def _():
                input_start(par, chunk_off(u + 2 * _NW), _CHK)

        for par in range(2):
            uu = wid + par * _NW

            @pl.when(uu < nchunks)
            def _():
                input_start(par, chunk_off(uu), _CHK)

        def body(k2, carry):
            for par in range(2):
                k = 2 * k2 + par
                u = wid + k * _NW

                @pl.when(u < nchunks)
                def _():
                    process_chunk(par, k, u)
            return carry

        lax.fori_loop(0, (max_k + 1) // 2, body, 0)
        for sp in range(4):
            drain(sp, K2)

    def do_terminals(start, count, K, scale):
        nblocks = (count + _BLK - 1) // _BLK
        range_end = start + count
        nunits = nblocks * K
        max_nu = (nunits + _NW - 1) // _NW

        def stage_vreg(par, off, off_orig, i_tap, v):
            xb, yb, sxb, syb = inbufs[par]
            idxb, valb = stag[par]
            sl = pl.ds(v * 16, 16)
            x = xb[sl]
            y = yb[sl]
            xe = x + sxb[sl]
            ye = y + syb[sl]
            bx0 = (x * 0.5).astype(jnp.int32)
            by0 = (y * 0.5).astype(jnp.int32)
            blx = bx0.astype(jnp.float32) * 2.0
            bly = by0.astype(jnp.float32) * 2.0
            ibase = bx0 * _NBY + by0
            lane = lax.iota(jnp.int32, 16)
            gid = off + v * 16 + lane
            mask = jnp.logical_and(gid >= off_orig, gid < range_end)
            fi = i_tap.astype(jnp.float32)
            lx = blx + 2.0 * fi
            oxi = jnp.maximum(
                jnp.minimum(xe, lx + 2.0) - jnp.maximum(x, lx), 0.0) * scale
            oxi = jnp.where(mask, oxi, 0.0)
            ibase = jnp.where(mask, ibase, v * 16 + lane) + i_tap * _NBY
            oy = overlaps(bly, y, ye, K, 1.0, None)
            for j in range(K):
                idxb[j, sl] = ibase + j
                valb[j, sl] = oxi * oy[j]

        def process_unit(par, u):
            b = u // K
            i_tap = u - b * K
            off_orig = start + b * _BLK
            off = jnp.minimum(off_orig, range_end - _BLK)
            input_wait(par, off, _BLK)

            def vbody(v, carry):
                stage_vreg(par, off, off_orig, i_tap, v)
                return carry

            lax.fori_loop(0, _BLK // 16, vbody, 0)

            @pl.when(u + 2 * _NW < nunits)
            def _():
                un = u + 2 * _NW
                bn = un // K
                input_start(
                    par,
                    jnp.minimum(start + bn * _BLK, range_end - _BLK), _BLK)

            fire(par, K)

        for par in range(2):
            uu = wid + par * _NW

            @pl.when(uu < nunits)
            def _():
                bb = uu // K
                input_start(
                    par,
                    jnp.minimum(start + bb * _BLK, range_end - _BLK), _BLK)

        def body(k2, carry):
            for par in range(2):
                k = 2 * k2 + par
                u = wid + k * _NW

                @pl.when(jnp.logical_and(u >= 2 * _NW, u < nunits + 2 * _NW))
                def _():
                    drain(par, K)

                @pl.when(u < nunits)
                def _():
                    process_unit(par, u)
            return carry

        lax.fori_loop(0, (max_nu + 2 + 1) // 2, body, 0)

    do_range(0, _NM, 3, 1.0)
    do_terminals(_NM, _NT, 9, _TD)
    do_range(_NM + _NT, _NF, 2, 1.0)

    plsc.subcore_barrier()
    pltpu.sync_copy(map_sh.at[pl.ds(sid * _LPT, _LPT)], zb)
    pltpu.sync_copy(zb, out_hbm.at[cid, pl.ds(sid * _LPT, _LPT)])


_scatter_maps = functools.partial(
    pl.kernel,
    out_type=jax.ShapeDtypeStruct((2, _NBINS), jnp.float32),
    mesh=plsc.VectorSubcoreMesh(core_axis_name="c", subcore_axis_name="s"),
    scratch_types=[
        pltpu.VMEM((_CHK,), jnp.float32),
        pltpu.VMEM((_CHK,), jnp.float32),
        pltpu.VMEM((_CHK,), jnp.float32),
        pltpu.VMEM((_CHK,), jnp.float32),
        pltpu.VMEM((_CHK,), jnp.float32),
        pltpu.VMEM((_CHK,), jnp.float32),
        pltpu.VMEM((_CHK,), jnp.float32),
        pltpu.VMEM((_CHK,), jnp.float32),
        pltpu.VMEM((9, _BLK), jnp.int32),
        pltpu.VMEM((9, _BLK), jnp.float32),
        pltpu.VMEM((9, _BLK), jnp.int32),
        pltpu.VMEM((9, _BLK), jnp.float32),
        pltpu.VMEM((9, _BLK), jnp.int32),
        pltpu.VMEM((9, _BLK), jnp.float32),
        pltpu.VMEM((9, _BLK), jnp.int32),
        pltpu.VMEM((9, _BLK), jnp.float32),
        pltpu.VMEM((_LPT,), jnp.float32),
        pltpu.VMEM_SHARED((_NBINS,), jnp.float32),
        pltpu.SemaphoreType.DMA,
        pltpu.SemaphoreType.DMA,
        pltpu.SemaphoreType.DMA,
        pltpu.SemaphoreType.DMA,
        pltpu.SemaphoreType.DMA,
        pltpu.SemaphoreType.DMA,
    ],
)(_sc_body)


def _reduce_body(maps_ref, cost_ref, md_ref):
    m = maps_ref[0] + maps_ref[1]
    ov = jnp.maximum(m - _TD * 4.0, 0.0)
    cost_ref[0, 0] = jnp.sum(ov)
    md_ref[0, 0] = jnp.max(m) * 0.25


def kernel(pos, node_size_x, node_size_y, bin_center_x, bin_center_y):
    maps = _scatter_maps(pos, node_size_x, node_size_y)
    maps = maps.reshape(2, 512, 512)
    cost, md = pl.pallas_call(
        _reduce_body,
        out_shape=(jax.ShapeDtypeStruct((1, 1), jnp.float32),
                   jax.ShapeDtypeStruct((1, 1), jnp.float32)),
        in_specs=[pl.BlockSpec(memory_space=pltpu.VMEM)],
        out_specs=(pl.BlockSpec(memory_space=pltpu.SMEM),
                   pl.BlockSpec(memory_space=pltpu.SMEM)),
    )(maps)
    return cost[0, 0], md[0, 0]


# speedup vs baseline: 1.0808x; 1.0808x over previous
import functools

import jax
import jax.numpy as jnp
from jax import lax
from jax.experimental import pallas as pl
from jax.experimental.pallas import tpu as pltpu
from jax.experimental.pallas import tpu_sc as plsc

_NM, _NT, _NF = 400000, 10000, 100000
_N = _NM + _NT + _NF
_NBY = 512
_NBINS = 512 * 512
_TD = 0.9
_BLK = 128
_NW = 32
_LPT = _NBINS // 16


def _sc_body(pos_hbm, nsx_hbm, nsy_hbm, out_hbm,
             xb0, yb0, sxb0, syb0, xb1, yb1, sxb1, syb1,
             idx0, val0, idx1, val1, zb, wdrain, map_sh,
             insem0, insem1, scsem0, scsem1):
    cid = lax.axis_index("c")
    sid = lax.axis_index("s")
    wid = sid * 2 + cid
    inbufs = ((xb0, yb0, sxb0, syb0), (xb1, yb1, sxb1, syb1))
    stag = ((idx0, val0), (idx1, val1))
    insems = (insem0, insem1)
    scsems = (scsem0, scsem1)

    def zero_body(i, carry):
        zb[pl.ds(i * 16, 16)] = jnp.zeros((16,), jnp.float32)
        return carry

    lax.fori_loop(0, _LPT // 16, zero_body, 0)

    pltpu.sync_copy(zb, map_sh.at[pl.ds(sid * _LPT, _LPT)])
    plsc.subcore_barrier()

    def do_range(start, count, K, scale, split_taps=False):
        nblocks = (count + _BLK - 1) // _BLK
        range_end = start + count
        clamp_off = range_end - _BLK
        nrows = K if split_taps else K * K
        nunits = nblocks * K if split_taps else nblocks
        max_nu = (nunits + _NW - 1) // _NW

        def unit_block(u):
            return u // K if split_taps else u

        def block_off(b):
            return jnp.minimum(start + b * _BLK, clamp_off)

        def input_start(par, off):
            xb, yb, sxb, syb = inbufs[par]
            sem = insems[par]
            pltpu.async_copy(pos_hbm.at[pl.ds(off, _BLK)], xb, sem)
            pltpu.async_copy(pos_hbm.at[pl.ds(_N + off, _BLK)], yb, sem)
            pltpu.async_copy(nsx_hbm.at[pl.ds(off, _BLK)], sxb, sem)
            pltpu.async_copy(nsy_hbm.at[pl.ds(off, _BLK)], syb, sem)

        def input_wait(par, off):
            sem = insems[par]
            pltpu.make_async_copy(
                pos_hbm.at[pl.ds(0, 4 * _BLK)], wdrain, sem).wait()

        def drain_scatters(par):
            idxb, valb = stag[par]
            sem = scsems[par]
            for t in range(nrows):
                pltpu.make_async_copy(
                    valb.at[t], map_sh.at[idxb.at[t]], sem).wait()

        def fire_scatters(par):
            idxb, valb = stag[par]
            sem = scsems[par]
            for t in range(nrows):
                pltpu.async_copy(
                    valb.at[t], map_sh.at[idxb.at[t]], sem, add=True)

        def stage_vreg(par, off, off_orig, i_tap, v):
            xb, yb, sxb, syb = inbufs[par]
            idxb, valb = stag[par]
            sl = pl.ds(v * 16, 16)
            x = xb[sl]
            y = yb[sl]
            xe = x + sxb[sl]
            ye = y + syb[sl]
            bx0 = (x * 0.5).astype(jnp.int32)
            by0 = (y * 0.5).astype(jnp.int32)
            blx = bx0.astype(jnp.float32) * 2.0
            bly = by0.astype(jnp.float32) * 2.0
            ibase = bx0 * _NBY + by0
            if count % _BLK == 0:
                mask = None
            else:
                lane = lax.iota(jnp.int32, 16)
                gid = off + v * 16 + lane
                mask = jnp.logical_and(gid >= off_orig, gid < range_end)
                dummy = v * 16 + lane
            oy = []
            for j in range(K):
                ly = bly + (2.0 * j)
                oy.append(jnp.maximum(
                    jnp.minimum(ye, ly + 2.0) - jnp.maximum(y, ly), 0.0))
            if split_taps:
                fi = i_tap.astype(jnp.float32)
                lx = blx + 2.0 * fi
                oxi = jnp.maximum(
                    jnp.minimum(xe, lx + 2.0) - jnp.maximum(x, lx), 0.0)
                if mask is not None:
                    oxi = jnp.where(mask, oxi, 0.0)
                if mask is not None:
                    ibase = jnp.where(mask, ibase, dummy)
                ibase = ibase + i_tap * _NBY
                oxs = oxi if scale == 1.0 else oxi * scale
                for j in range(K):
                    idxb[j, sl] = ibase + j
                    valb[j, sl] = oxs * oy[j]
            else:
                ox = []
                for i in range(K):
                    lx = blx + (2.0 * i)
                    oxi = jnp.maximum(
                        jnp.minimum(xe, lx + 2.0) - jnp.maximum(x, lx), 0.0)
                    if mask is not None:
                        oxi = jnp.where(mask, oxi, 0.0)
                    ox.append(oxi)
                if mask is not None:
                    ibase = jnp.where(mask, ibase, dummy)
                for i in range(K):
                    oxs = ox[i] if scale == 1.0 else ox[i] * scale
                    for j in range(K):
                        t = i * K + j
                        idxb[t, sl] = ibase + (i * _NBY + j)
                        valb[t, sl] = oxs * oy[j]

        def process_unit(par, u):
            b = unit_block(u)
            i_tap = (u - b * K) if split_taps else 0
            off = block_off(b)
            off_orig = start + b * _BLK
            input_wait(par, off)

            def vbody(v, carry):
                stage_vreg(par, off, off_orig, i_tap, v)
                return carry

            lax.fori_loop(0, _BLK // 16, vbody, 0)

            @pl.when(u + 2 * _NW < nunits)
            def _():
                input_start(par, block_off(unit_block(u + 2 * _NW)))

            fire_scatters(par)

        for par in range(2):
            uu = wid + par * _NW

            @pl.when(uu < nunits)
            def _():
                input_start(par, block_off(unit_block(uu)))

        def body(k2, carry):
            for par in range(2):
                k = 2 * k2 + par
                u = wid + k * _NW

                @pl.when(jnp.logical_and(u >= 2 * _NW, u < nunits + 2 * _NW))
                def _():
                    drain_scatters(par)

                @pl.when(u < nunits)
                def _():
                    process_unit(par, u)
            return carry

        lax.fori_loop(0, (max_nu + 2 + 1) // 2, body, 0)

    do_range(0, _NM, 3, 1.0)
    do_range(_NM, _NT, 9, _TD, split_taps=True)
    do_range(_NM + _NT, _NF, 2, 1.0)

    plsc.subcore_barrier()
    pltpu.sync_copy(map_sh.at[pl.ds(sid * _LPT, _LPT)], zb)
    pltpu.sync_copy(zb, out_hbm.at[cid, pl.ds(sid * _LPT, _LPT)])


_scatter_maps = functools.partial(
    pl.kernel,
    out_type=jax.ShapeDtypeStruct((2, _NBINS), jnp.float32),
    mesh=plsc.VectorSubcoreMesh(core_axis_name="c", subcore_axis_name="s"),
    scratch_types=[
        pltpu.VMEM((_BLK,), jnp.float32),
        pltpu.VMEM((_BLK,), jnp.float32),
        pltpu.VMEM((_BLK,), jnp.float32),
        pltpu.VMEM((_BLK,), jnp.float32),
        pltpu.VMEM((_BLK,), jnp.float32),
        pltpu.VMEM((_BLK,), jnp.float32),
        pltpu.VMEM((_BLK,), jnp.float32),
        pltpu.VMEM((_BLK,), jnp.float32),
        pltpu.VMEM((9, _BLK), jnp.int32),
        pltpu.VMEM((9, _BLK), jnp.float32),
        pltpu.VMEM((9, _BLK), jnp.int32),
        pltpu.VMEM((9, _BLK), jnp.float32),
        pltpu.VMEM((_LPT,), jnp.float32),
        pltpu.VMEM((4 * _BLK,), jnp.float32),
        pltpu.VMEM_SHARED((_NBINS,), jnp.float32),
        pltpu.SemaphoreType.DMA,
        pltpu.SemaphoreType.DMA,
        pltpu.SemaphoreType.DMA,
        pltpu.SemaphoreType.DMA,
    ],
)(_sc_body)


def _reduce_body(maps_ref, cost_ref, md_ref):
    m = maps_ref[0] + maps_ref[1]
    ov = jnp.maximum(m - _TD * 4.0, 0.0)
    cost_ref[0, 0] = jnp.sum(ov)
    md_ref[0, 0] = jnp.max(m) * 0.25


def kernel(pos, node_size_x, node_size_y, bin_center_x, bin_center_y):
    maps = _scatter_maps(pos, node_size_x, node_size_y)
    maps = maps.reshape(2, 512, 512)
    cost, md = pl.pallas_call(
        _reduce_body,
        out_shape=(jax.ShapeDtypeStruct((1, 1), jnp.float32),
                   jax.ShapeDtypeStruct((1, 1), jnp.float32)),
        in_specs=[pl.BlockSpec(memory_space=pltpu.VMEM)],
        out_specs=(pl.BlockSpec(memory_space=pltpu.SMEM),
                   pl.BlockSpec(memory_space=pltpu.SMEM)),
    )(maps)
    return cost[0, 0], md[0, 0]
